# baseline (speedup 1.0000x reference)
import functools

import jax
import jax.numpy as jnp
from jax import lax
from jax.experimental import pallas as pl
from jax.experimental.pallas import tpu as pltpu
from jax.experimental.pallas import tpu_sc as plsc

BATCH = 16384
FEAT = 128
NCLS = 1000
NCLS_P = 1024
MOM = 0.99

NC = 2
NS = 16
NW = NC * NS
RPW = BATCH // NW
CH = 128
NCHUNK = RPW // CH
CROWS = NCLS_P // NS

P2_BLK = 2048
P2_STEPS = BATCH // P2_BLK


def _rsqrt_newton(s2):
    s2 = jnp.maximum(s2, 1e-24)
    m = lax.bitcast_convert_type(s2, jnp.int32)
    y = lax.bitcast_convert_type(
        jnp.int32(0x5F3759DF) - lax.shift_right_logical(m, 1), jnp.float32)
    for _ in range(2):
        y = y * (1.5 - 0.5 * s2 * y * y)
    return y


def _sc_segsum_body(feat_hbm, lab_hbm, zsum_hbm, zcnt_hbm, ones_hbm,
                    sums_out, cnt_out,
                    fchunk0, fchunk1, lchunk0, lchunk1, ones_v, acc_s, cnt_s,
                    fsems, lsems):
    fchunks = [fchunk0, fchunk1]
    lchunks = [lchunk0, lchunk1]
    cid = lax.axis_index("c")
    sid = lax.axis_index("s")
    base = (sid * NC + cid) * RPW

    pltpu.sync_copy(zsum_hbm.at[pl.ds(sid * CROWS, CROWS)],
                    acc_s.at[pl.ds(sid * CROWS, CROWS)])
    pltpu.sync_copy(zcnt_hbm.at[pl.ds(sid * CROWS, CROWS)],
                    cnt_s.at[pl.ds(sid * CROWS, CROWS)])
    pltpu.sync_copy(ones_hbm, ones_v)
    plsc.subcore_barrier()

    for t in range(NCHUNK):
        off = base + t * CH
        fchunk = fchunks[t % 2]
        pltpu.sync_copy(feat_hbm.at[pl.ds(off, CH)], fchunk)
        pltpu.sync_copy(lab_hbm.at[pl.ds(off, CH)], lchunks[t % 2])

        def row_body(r, carry):
            acc = jnp.zeros((16,), jnp.float32)
            for k in range(FEAT // 16):
                v = fchunk[r, pl.ds(k * 16, 16)]
                acc = acc + v * v
            inv = _rsqrt_newton(jnp.full((16,), jnp.sum(acc), jnp.float32))
            for k in range(FEAT // 16):
                fchunk[r, pl.ds(k * 16, 16)] = fchunk[r, pl.ds(k * 16, 16)] * inv
            return carry

        lax.fori_loop(0, CH, row_body, 0)
        pltpu.sync_copy(fchunk, acc_s.at[lchunks[t % 2]], add=True)
        pltpu.sync_copy(ones_v, cnt_s.at[lchunks[t % 2]], add=True)

    plsc.subcore_barrier()
    pltpu.sync_copy(acc_s.at[pl.ds(sid * CROWS, CROWS)],
                    sums_out.at[cid, pl.ds(sid * CROWS, CROWS)])
    pltpu.sync_copy(cnt_s.at[pl.ds(sid * CROWS, CROWS)],
                    cnt_out.at[cid, pl.ds(sid * CROWS, CROWS)])


@functools.lru_cache(maxsize=1)
def _sc_segsum():
    return pl.kernel(
        _sc_segsum_body,
        out_type=(
            jax.ShapeDtypeStruct((NC, NCLS_P, FEAT), jnp.float32),
            jax.ShapeDtypeStruct((NC, NCLS_P, 16), jnp.float32),
        ),
        mesh=plsc.VectorSubcoreMesh(core_axis_name="c", subcore_axis_name="s"),
        compiler_params=pltpu.CompilerParams(needs_layout_passes=False),
        scratch_types=[
            pltpu.VMEM((CH, FEAT), jnp.float32),
            pltpu.VMEM((CH, FEAT), jnp.float32),
            pltpu.VMEM((CH,), jnp.int32),
            pltpu.VMEM((CH,), jnp.int32),
            pltpu.VMEM((CH, 16), jnp.float32),
            pltpu.VMEM_SHARED((NCLS_P, FEAT), jnp.float32),
            pltpu.VMEM_SHARED((NCLS_P, 16), jnp.float32),
            pltpu.SemaphoreType.DMA((NCHUNK,)),
            pltpu.SemaphoreType.DMA((NCHUNK,)),
        ],
    )


def _norm_rows(x):
    s2 = jnp.sum(x * x, axis=1, keepdims=True)
    return x * lax.rsqrt(jnp.maximum(s2, 1e-24))


def _tc_body(f_ref, protos_ref, sums_ref, cnt_ref, out_ref, pbf_ref):
    i = pl.program_id(0)

    @pl.when(i == 0)
    def _make_protos():
        sums = sums_ref[0, :NCLS, :] + sums_ref[1, :NCLS, :]
        counts = cnt_ref[0, :NCLS, :1] + cnt_ref[1, :NCLS, :1]
        p0 = _norm_rows(protos_ref[...])
        sn = _norm_rows(sums)
        bl = _norm_rows(MOM * p0 + (1.0 - MOM) * sn)
        p = jnp.where(counts > 0.0, bl, p0)
        pbf_ref[...] = p.astype(jnp.bfloat16)

    fn = _norm_rows(f_ref[...]).astype(jnp.bfloat16)
    s = lax.dot_general(
        fn, pbf_ref[...], (((1,), (1,)), ((), ())),
        preferred_element_type=jnp.float32)
    out_ref[...] = -jnp.sqrt(jnp.maximum(2.0 - 2.0 * s, 0.0))


def kernel(features, labels, prototypes):
    zsum = jnp.zeros((NCLS_P, FEAT), jnp.float32)
    zcnt = jnp.zeros((NCLS_P, 16), jnp.float32)
    ones = jnp.ones((CH, 16), jnp.float32)
    sums2, cnt2 = _sc_segsum()(features, labels, zsum, zcnt, ones)

    out = pl.pallas_call(
        _tc_body,
        grid=(P2_STEPS,),
        in_specs=[
            pl.BlockSpec((P2_BLK, FEAT), lambda i: (i, 0)),
            pl.BlockSpec((NCLS, FEAT), lambda i: (0, 0)),
            pl.BlockSpec((NC, NCLS_P, FEAT), lambda i: (0, 0, 0)),
            pl.BlockSpec((NC, NCLS_P, 16), lambda i: (0, 0, 0)),
        ],
        out_specs=pl.BlockSpec((P2_BLK, NCLS), lambda i: (i, 0)),
        out_shape=jax.ShapeDtypeStruct((BATCH, NCLS), jnp.float32),
        scratch_shapes=[pltpu.VMEM((NCLS, FEAT), jnp.bfloat16)],
    )(features, prototypes, sums2, cnt2)
    return out


# speedup vs baseline: 1.0896x; 1.0896x over previous
import functools

import jax
import jax.numpy as jnp
from jax import lax
from jax.experimental import pallas as pl
from jax.experimental.pallas import tpu as pltpu
from jax.experimental.pallas import tpu_sc as plsc

BATCH = 16384
FEAT = 128
NCLS = 1000
NCLS_P = 1024
MOM = 0.99

NC = 2
NS = 16
NW = NC * NS
RPW = BATCH // NW
CH = 128
NCHUNK = RPW // CH
CROWS = NCLS_P // NS

P2_BLK = 2048
P2_STEPS = BATCH // P2_BLK


def _rsqrt_newton(s2):
    s2 = jnp.maximum(s2, 1e-24)
    m = lax.bitcast_convert_type(s2, jnp.int32)
    y = lax.bitcast_convert_type(
        jnp.int32(0x5F3759DF) - lax.shift_right_logical(m, 1), jnp.float32)
    for _ in range(2):
        y = y * (1.5 - 0.5 * s2 * y * y)
    return y


def _sc_segsum_body(feat_hbm, lab_hbm, zsum_hbm, zcnt_hbm, ones_hbm,
                    sums_out, cnt_out,
                    fchunk0, fchunk1, lchunk0, lchunk1, ones_v, acc_s, cnt_s,
                    fsems, lsems):
    fchunks = [fchunk0, fchunk1]
    lchunks = [lchunk0, lchunk1]
    cid = lax.axis_index("c")
    sid = lax.axis_index("s")
    base = (sid * NC + cid) * RPW

    pltpu.sync_copy(zsum_hbm.at[pl.ds(sid * CROWS, CROWS)],
                    acc_s.at[pl.ds(sid * CROWS, CROWS)])
    pltpu.sync_copy(zcnt_hbm.at[pl.ds(sid * CROWS, CROWS)],
                    cnt_s.at[pl.ds(sid * CROWS, CROWS)])
    pltpu.sync_copy(ones_hbm, ones_v)
    plsc.subcore_barrier()

    fchunk, nchunk = fchunk0, fchunk1
    for t in range(NCHUNK):
        off = base + t * CH
        pltpu.sync_copy(feat_hbm.at[pl.ds(off, CH)], fchunk)
        pltpu.sync_copy(lab_hbm.at[pl.ds(off, CH)], lchunk0)

        def row_body(r, carry):
            acc = jnp.zeros((16,), jnp.float32)
            for k in range(FEAT // 16):
                v = fchunk[r, pl.ds(k * 16, 16)]
                acc = acc + v * v
            inv = _rsqrt_newton(jnp.full((16,), jnp.sum(acc), jnp.float32))
            for k in range(FEAT // 16):
                nchunk[r, pl.ds(k * 16, 16)] = fchunk[r, pl.ds(k * 16, 16)] * inv
            return carry

        lax.fori_loop(0, CH, row_body, 0)
        pltpu.sync_copy(nchunk, acc_s.at[lchunk0], add=True)
        pltpu.sync_copy(ones_v, cnt_s.at[lchunk0], add=True)

    plsc.subcore_barrier()
    pltpu.sync_copy(acc_s.at[pl.ds(sid * CROWS, CROWS)],
                    sums_out.at[cid, pl.ds(sid * CROWS, CROWS)])
    pltpu.sync_copy(cnt_s.at[pl.ds(sid * CROWS, CROWS)],
                    cnt_out.at[cid, pl.ds(sid * CROWS, CROWS)])


@functools.lru_cache(maxsize=1)
def _sc_segsum():
    return pl.kernel(
        _sc_segsum_body,
        out_type=(
            jax.ShapeDtypeStruct((NC, NCLS_P, FEAT), jnp.float32),
            jax.ShapeDtypeStruct((NC, NCLS_P, 16), jnp.float32),
        ),
        mesh=plsc.VectorSubcoreMesh(core_axis_name="c", subcore_axis_name="s"),
        compiler_params=pltpu.CompilerParams(needs_layout_passes=False),
        scratch_types=[
            pltpu.VMEM((CH, FEAT), jnp.float32),
            pltpu.VMEM((CH, FEAT), jnp.float32),
            pltpu.VMEM((CH,), jnp.int32),
            pltpu.VMEM((CH,), jnp.int32),
            pltpu.VMEM((CH, 16), jnp.float32),
            pltpu.VMEM_SHARED((NCLS_P, FEAT), jnp.float32),
            pltpu.VMEM_SHARED((NCLS_P, 16), jnp.float32),
            pltpu.SemaphoreType.DMA((NCHUNK,)),
            pltpu.SemaphoreType.DMA((NCHUNK,)),
        ],
    )


def _norm_rows(x):
    s2 = jnp.sum(x * x, axis=1, keepdims=True)
    return x * lax.rsqrt(jnp.maximum(s2, 1e-24))


def _tc_body(f_ref, protos_ref, sums_ref, cnt_ref, out_ref, pbf_ref):
    i = pl.program_id(0)

    @pl.when(i == 0)
    def _make_protos():
        sums = sums_ref[0, :NCLS, :] + sums_ref[1, :NCLS, :]
        counts = cnt_ref[0, :NCLS, :1] + cnt_ref[1, :NCLS, :1]
        p0 = _norm_rows(protos_ref[...])
        sn = _norm_rows(sums)
        bl = _norm_rows(MOM * p0 + (1.0 - MOM) * sn)
        p = jnp.where(counts > 0.0, bl, p0)
        pbf_ref[...] = p.astype(jnp.bfloat16)

    fn = _norm_rows(f_ref[...]).astype(jnp.bfloat16)
    s = lax.dot_general(
        fn, pbf_ref[...], (((1,), (1,)), ((), ())),
        preferred_element_type=jnp.float32)
    out_ref[...] = -jnp.sqrt(jnp.maximum(2.0 - 2.0 * s, 0.0))


def kernel(features, labels, prototypes):
    zsum = jnp.zeros((NCLS_P, FEAT), jnp.float32)
    zcnt = jnp.zeros((NCLS_P, 16), jnp.float32)
    ones = jnp.ones((CH, 16), jnp.float32)
    sums2, cnt2 = _sc_segsum()(features, labels, zsum, zcnt, ones)

    out = pl.pallas_call(
        _tc_body,
        grid=(P2_STEPS,),
        in_specs=[
            pl.BlockSpec((P2_BLK, FEAT), lambda i: (i, 0)),
            pl.BlockSpec((NCLS, FEAT), lambda i: (0, 0)),
            pl.BlockSpec((NC, NCLS_P, FEAT), lambda i: (0, 0, 0)),
            pl.BlockSpec((NC, NCLS_P, 16), lambda i: (0, 0, 0)),
        ],
        out_specs=pl.BlockSpec((P2_BLK, NCLS), lambda i: (i, 0)),
        out_shape=jax.ShapeDtypeStruct((BATCH, NCLS), jnp.float32),
        scratch_shapes=[pltpu.VMEM((NCLS, FEAT), jnp.bfloat16)],
    )(features, prototypes, sums2, cnt2)
    return out
